# baseline (speedup 1.0000x reference)
import functools

import jax
import jax.numpy as jnp
from jax.experimental import pallas as pl
from jax.experimental.pallas import tpu as pltpu

HEADC = 128


def _band_weights(wk, w):
    cin, cout = wk.shape[1], wk.shape[2]
    bands = jnp.zeros((3, w, cin, w, cout), wk.dtype)
    for dh in range(3):
        for dw in range(3):
            for wo in range(w):
                wi = wo + dw - 1
                if 0 <= wi < w:
                    bands = bands.at[dh, wi, :, wo, :].set(wk[dh * 3 + dw])
    return bands.reshape(3, w * cin, w * cout).astype(jnp.bfloat16)


def _fused_kernel(x_ref, cb1_ref, b1_ref, cb2_ref, b2_ref, cb3_ref, b3_ref,
                  hdw_ref, hdb_ref, wbig_ref, pb2_ref, vb2_ref, vw3t_ref,
                  vb3_ref, prob_ref, val_ref, pad1, pad2, *, nb, h, w):
    mh = h * nb

    @pl.when(pl.program_id(0) == 0)
    def _():
        for ref in (pad1, pad2):
            ref[0, :, :] = jnp.zeros_like(ref[0, :, :])
            ref[h + 1, :, :] = jnp.zeros_like(ref[h + 1, :, :])

    def conv3x3_relu(src_ref, band_ref, b_ref):
        acc = None
        for dh in range(3):
            part = jnp.dot(src_ref[pl.ds(dh, h), :, :].reshape(mh, -1),
                           band_ref[dh], preferred_element_type=jnp.float32)
            acc = part if acc is None else acc + part
        return jnp.maximum(acc + b_ref[...], 0.0)

    y1 = conv3x3_relu(x_ref, cb1_ref, b1_ref).astype(jnp.bfloat16)

    pad1[pl.ds(1, h), :, :] = y1.reshape(h, nb, -1)
    y2 = conv3x3_relu(pad1, cb2_ref, b2_ref).astype(jnp.bfloat16)

    pad2[pl.ds(1, h), :, :] = y2.reshape(h, nb, -1)
    y3 = conv3x3_relu(pad2, cb3_ref, b3_ref).astype(jnp.bfloat16)

    heads = jnp.dot(y3, hdw_ref[...], preferred_element_type=jnp.float32)
    heads = jnp.maximum(heads + hdb_ref[...], 0.0).astype(jnp.bfloat16)

    hs = jnp.swapaxes(heads.reshape(h, nb, w * HEADC), 0, 1)
    hv = jnp.dot(hs.reshape(nb, h * w * HEADC), wbig_ref[...],
                 preferred_element_type=jnp.float32)

    logits = hv[:, : h * w] + pb2_ref[...]
    mx = jnp.max(logits, axis=-1, keepdims=True)
    s = logits - mx
    lse = jnp.log(jnp.sum(jnp.exp(s), axis=-1, keepdims=True))
    prob_ref[...] = (s - lse).astype(prob_ref.dtype)

    v = jnp.maximum(hv[:, h * w: h * w + 64] + vb2_ref[...], 0.0)
    val = jnp.sum(v * vw3t_ref[...], axis=-1, keepdims=True) + vb3_ref[...]
    val_ref[...] = jnp.tanh(val).astype(val_ref.dtype)


def kernel(x_nchw, conv_w1, conv_w2, conv_w3, conv_b1, conv_b2, conv_b3,
           head_w, head_b, pw2, pb2, vw2, vb2, vw3, vb3):
    n, c, h, w = x_nchw.shape
    hw = h * w
    nb = next(cand for cand in (96, 32, 16, 8, 4, 2, 1) if n % cand == 0)
    bf = jnp.bfloat16

    xa = jnp.transpose(x_nchw.reshape(n, c, hw), (0, 2, 1)).astype(bf)
    xa = jax.lax.optimization_barrier(xa)
    x = jnp.pad(jnp.transpose(xa.reshape(n, h, w, c), (1, 0, 2, 3))
                .reshape(h, n, w * c), ((1, 1), (0, 0), (0, 0)))

    cb1 = _band_weights(conv_w1, w)
    cb2 = _band_weights(conv_w2, w)
    cb3 = _band_weights(conv_w3, w)
    b1 = jnp.tile(conv_b1, (1, w))
    b2 = jnp.tile(conv_b2, (1, w))
    b3 = jnp.tile(conv_b3, (1, w))

    hd = jnp.zeros((w, 128, w, HEADC), jnp.float32)
    for i in range(w):
        hd = hd.at[i, :, i, :].set(head_w)
    hd = hd.reshape(w * 128, w * HEADC).astype(bf)
    hb = jnp.tile(head_b, (1, w))

    hw_out = pw2.shape[1]
    big = jnp.zeros((hw, HEADC, hw_out + 64), jnp.float32)
    big = big.at[:, :4, :hw_out].set(pw2.reshape(hw, 4, hw_out))
    big = big.at[:, 4:6, hw_out:].set(vw2.reshape(hw, 2, 64))
    wbig = big.reshape(hw * HEADC, hw_out + 64).astype(bf)

    vw3t = vw3.reshape(1, -1)

    fused = functools.partial(_fused_kernel, nb=nb, h=h, w=w)
    prob_out, val_out = pl.pallas_call(
        fused,
        out_shape=(jax.ShapeDtypeStruct((n, hw_out), jnp.float32),
                   jax.ShapeDtypeStruct((n, 1), jnp.float32)),
        grid=(n // nb,),
        in_specs=[
            pl.BlockSpec((h + 2, nb, w * c), lambda b: (0, b, 0)),
            pl.BlockSpec(cb1.shape, lambda b: (0, 0, 0)),
            pl.BlockSpec(b1.shape, lambda b: (0, 0)),
            pl.BlockSpec(cb2.shape, lambda b: (0, 0, 0)),
            pl.BlockSpec(b2.shape, lambda b: (0, 0)),
            pl.BlockSpec(cb3.shape, lambda b: (0, 0, 0)),
            pl.BlockSpec(b3.shape, lambda b: (0, 0)),
            pl.BlockSpec(hd.shape, lambda b: (0, 0)),
            pl.BlockSpec(hb.shape, lambda b: (0, 0)),
            pl.BlockSpec(wbig.shape, lambda b: (0, 0)),
            pl.BlockSpec(pb2.shape, lambda b: (0, 0)),
            pl.BlockSpec(vb2.shape, lambda b: (0, 0)),
            pl.BlockSpec(vw3t.shape, lambda b: (0, 0)),
            pl.BlockSpec(vb3.shape, lambda b: (0, 0)),
        ],
        out_specs=(pl.BlockSpec((nb, hw_out), lambda b: (b, 0)),
                   pl.BlockSpec((nb, 1), lambda b: (b, 0))),
        scratch_shapes=[
            pltpu.VMEM((h + 2, nb, w * 32), bf),
            pltpu.VMEM((h + 2, nb, w * 64), bf),
        ],
        compiler_params=pltpu.CompilerParams(
            dimension_semantics=("arbitrary",)),
    )(x, cb1, b1, cb2, b2, cb3, b3, hd, hb, wbig, pb2, vb2, vw3t, vb3)
    return prob_out, val_out


# speedup vs baseline: 1.0289x; 1.0289x over previous
import functools

import jax
import jax.numpy as jnp
from jax.experimental import pallas as pl
from jax.experimental.pallas import tpu as pltpu

HEADC = 128


def _band_weights(wk, w):
    cin, cout = wk.shape[1], wk.shape[2]
    bands = jnp.zeros((3, w, cin, w, cout), wk.dtype)
    for dh in range(3):
        for dw in range(3):
            for wo in range(w):
                wi = wo + dw - 1
                if 0 <= wi < w:
                    bands = bands.at[dh, wi, :, wo, :].set(wk[dh * 3 + dw])
    return bands.reshape(3, w * cin, w * cout).astype(jnp.bfloat16)


def _transpose_kernel(x_ref, out_ref, *, nbt, h, w):
    xb = x_ref[...].astype(jnp.bfloat16)
    xt = jnp.swapaxes(xb, 1, 2)
    c = xb.shape[1]
    o = jnp.transpose(xt.reshape(nbt, h, w, c), (1, 0, 2, 3))
    out_ref[0, :, :] = jnp.zeros_like(out_ref[0, :, :])
    out_ref[pl.ds(1, h), :, :] = o.reshape(h, nbt, w * c)
    out_ref[h + 1, :, :] = jnp.zeros_like(out_ref[h + 1, :, :])


def _nchw_to_padded(x_nchw, nbt):
    n, c, h, w = x_nchw.shape
    tk = functools.partial(_transpose_kernel, nbt=nbt, h=h, w=w)
    return pl.pallas_call(
        tk,
        out_shape=jax.ShapeDtypeStruct((h + 2, n, w * c), jnp.bfloat16),
        grid=(n // nbt,),
        in_specs=[pl.BlockSpec((nbt, c, h * w), lambda b: (b, 0, 0))],
        out_specs=pl.BlockSpec((h + 2, nbt, w * c), lambda b: (0, b, 0)),
        compiler_params=pltpu.CompilerParams(
            dimension_semantics=("arbitrary",)),
    )(x_nchw.reshape(n, c, h * w))


def _fused_kernel(x_ref, cb1_ref, b1_ref, cb2_ref, b2_ref, cb3_ref, b3_ref,
                  hdw_ref, hdb_ref, wbig_ref, pb2_ref, vb2_ref, vw3t_ref,
                  vb3_ref, prob_ref, val_ref, pad1, pad2, *, nb, h, w):
    mh = h * nb

    @pl.when(pl.program_id(0) == 0)
    def _():
        for ref in (pad1, pad2):
            ref[0, :, :] = jnp.zeros_like(ref[0, :, :])
            ref[h + 1, :, :] = jnp.zeros_like(ref[h + 1, :, :])

    def conv3x3_relu(src_ref, band_ref, b_ref):
        acc = None
        for dh in range(3):
            part = jnp.dot(src_ref[pl.ds(dh, h), :, :].reshape(mh, -1),
                           band_ref[dh], preferred_element_type=jnp.float32)
            acc = part if acc is None else acc + part
        return jnp.maximum(acc + b_ref[...], 0.0)

    y1 = conv3x3_relu(x_ref, cb1_ref, b1_ref).astype(jnp.bfloat16)

    pad1[pl.ds(1, h), :, :] = y1.reshape(h, nb, -1)
    y2 = conv3x3_relu(pad1, cb2_ref, b2_ref).astype(jnp.bfloat16)

    pad2[pl.ds(1, h), :, :] = y2.reshape(h, nb, -1)
    y3 = conv3x3_relu(pad2, cb3_ref, b3_ref).astype(jnp.bfloat16)

    heads = jnp.dot(y3, hdw_ref[...], preferred_element_type=jnp.float32)
    heads = jnp.maximum(heads + hdb_ref[...], 0.0).astype(jnp.bfloat16)

    hs = jnp.swapaxes(heads.reshape(h, nb, w * HEADC), 0, 1)
    hv = jnp.dot(hs.reshape(nb, h * w * HEADC), wbig_ref[...],
                 preferred_element_type=jnp.float32)

    logits = hv[:, : h * w] + pb2_ref[...]
    mx = jnp.max(logits, axis=-1, keepdims=True)
    s = logits - mx
    lse = jnp.log(jnp.sum(jnp.exp(s), axis=-1, keepdims=True))
    prob_ref[...] = (s - lse).astype(prob_ref.dtype)

    v = jnp.maximum(hv[:, h * w: h * w + 64] + vb2_ref[...], 0.0)
    val = jnp.sum(v * vw3t_ref[...], axis=-1, keepdims=True) + vb3_ref[...]
    val_ref[...] = jnp.tanh(val).astype(val_ref.dtype)


def kernel(x_nchw, conv_w1, conv_w2, conv_w3, conv_b1, conv_b2, conv_b3,
           head_w, head_b, pw2, pb2, vw2, vb2, vw3, vb3):
    n, c, h, w = x_nchw.shape
    hw = h * w
    nb = next(cand for cand in (96, 32, 16, 8, 4, 2, 1) if n % cand == 0)
    bf = jnp.bfloat16

    x = _nchw_to_padded(x_nchw, nb)

    cb1 = _band_weights(conv_w1, w)
    cb2 = _band_weights(conv_w2, w)
    cb3 = _band_weights(conv_w3, w)
    b1 = jnp.tile(conv_b1, (1, w))
    b2 = jnp.tile(conv_b2, (1, w))
    b3 = jnp.tile(conv_b3, (1, w))

    hd = jnp.zeros((w, 128, w, HEADC), jnp.float32)
    for i in range(w):
        hd = hd.at[i, :, i, :].set(head_w)
    hd = hd.reshape(w * 128, w * HEADC).astype(bf)
    hb = jnp.tile(head_b, (1, w))

    hw_out = pw2.shape[1]
    big = jnp.zeros((hw, HEADC, hw_out + 64), jnp.float32)
    big = big.at[:, :4, :hw_out].set(pw2.reshape(hw, 4, hw_out))
    big = big.at[:, 4:6, hw_out:].set(vw2.reshape(hw, 2, 64))
    wbig = big.reshape(hw * HEADC, hw_out + 64).astype(bf)

    vw3t = vw3.reshape(1, -1)

    fused = functools.partial(_fused_kernel, nb=nb, h=h, w=w)
    prob_out, val_out = pl.pallas_call(
        fused,
        out_shape=(jax.ShapeDtypeStruct((n, hw_out), jnp.float32),
                   jax.ShapeDtypeStruct((n, 1), jnp.float32)),
        grid=(n // nb,),
        in_specs=[
            pl.BlockSpec((h + 2, nb, w * c), lambda b: (0, b, 0)),
            pl.BlockSpec(cb1.shape, lambda b: (0, 0, 0)),
            pl.BlockSpec(b1.shape, lambda b: (0, 0)),
            pl.BlockSpec(cb2.shape, lambda b: (0, 0, 0)),
            pl.BlockSpec(b2.shape, lambda b: (0, 0)),
            pl.BlockSpec(cb3.shape, lambda b: (0, 0, 0)),
            pl.BlockSpec(b3.shape, lambda b: (0, 0)),
            pl.BlockSpec(hd.shape, lambda b: (0, 0)),
            pl.BlockSpec(hb.shape, lambda b: (0, 0)),
            pl.BlockSpec(wbig.shape, lambda b: (0, 0)),
            pl.BlockSpec(pb2.shape, lambda b: (0, 0)),
            pl.BlockSpec(vb2.shape, lambda b: (0, 0)),
            pl.BlockSpec(vw3t.shape, lambda b: (0, 0)),
            pl.BlockSpec(vb3.shape, lambda b: (0, 0)),
        ],
        out_specs=(pl.BlockSpec((nb, hw_out), lambda b: (b, 0)),
                   pl.BlockSpec((nb, 1), lambda b: (b, 0))),
        scratch_shapes=[
            pltpu.VMEM((h + 2, nb, w * 32), bf),
            pltpu.VMEM((h + 2, nb, w * 64), bf),
        ],
        compiler_params=pltpu.CompilerParams(
            dimension_semantics=("arbitrary",)),
    )(x, cb1, b1, cb2, b2, cb3, b3, hd, hb, wbig, pb2, vb2, vw3t, vb3)
    return prob_out, val_out


# speedup vs baseline: 1.0695x; 1.0395x over previous
import functools

import jax
import jax.numpy as jnp
from jax.experimental import pallas as pl
from jax.experimental.pallas import tpu as pltpu

HEADC = 128


def _band_weights(wk, w):
    cin, cout = wk.shape[1], wk.shape[2]
    bands = jnp.zeros((3, w, cin, w, cout), wk.dtype)
    for dh in range(3):
        for dw in range(3):
            for wo in range(w):
                wi = wo + dw - 1
                if 0 <= wi < w:
                    bands = bands.at[dh, wi, :, wo, :].set(wk[dh * 3 + dw])
    return bands.reshape(3, w * cin, w * cout).astype(jnp.bfloat16)


def _fused_kernel(x_ref, ident_ref, cb1_ref, b1_ref, cb2_ref, b2_ref,
                  cb3_ref, b3_ref, hdw_ref, hdb_ref, wbig_ref, pb2_ref,
                  vb2_ref, vw3t_ref, vb3_ref, prob_ref, val_ref,
                  pad0, pad1, pad2, *, nb, h, w):
    mh = h * nb

    @pl.when(pl.program_id(0) == 0)
    def _():
        for ref in (pad0, pad1, pad2):
            ref[0, :, :] = jnp.zeros_like(ref[0, :, :])
            ref[h + 1, :, :] = jnp.zeros_like(ref[h + 1, :, :])

    def conv3x3_relu(src_ref, band_ref, b_ref):
        acc = None
        for dh in range(3):
            part = jnp.dot(src_ref[pl.ds(dh, h), :, :].reshape(mh, -1),
                           band_ref[dh], preferred_element_type=jnp.float32)
            acc = part if acc is None else acc + part
        return jnp.maximum(acc + b_ref[...], 0.0)

    c = x_ref.shape[1]
    xb = x_ref[...].astype(jnp.bfloat16)
    xt = jax.lax.dot_general(xb, ident_ref[...], (((1,), (0,)), ((), ())),
                             preferred_element_type=jnp.float32)
    xt = xt.astype(jnp.bfloat16)
    o = jnp.transpose(xt.reshape(nb, h, w, c), (1, 0, 2, 3))
    pad0[pl.ds(1, h), :, :] = o.reshape(h, nb, w * c)
    y1 = conv3x3_relu(pad0, cb1_ref, b1_ref).astype(jnp.bfloat16)

    pad1[pl.ds(1, h), :, :] = y1.reshape(h, nb, -1)
    y2 = conv3x3_relu(pad1, cb2_ref, b2_ref).astype(jnp.bfloat16)

    pad2[pl.ds(1, h), :, :] = y2.reshape(h, nb, -1)
    y3 = conv3x3_relu(pad2, cb3_ref, b3_ref).astype(jnp.bfloat16)

    heads = jnp.dot(y3, hdw_ref[...], preferred_element_type=jnp.float32)
    heads = jnp.maximum(heads + hdb_ref[...], 0.0).astype(jnp.bfloat16)

    hs = jnp.swapaxes(heads.reshape(h, nb, w * HEADC), 0, 1)
    hv = jnp.dot(hs.reshape(nb, h * w * HEADC), wbig_ref[...],
                 preferred_element_type=jnp.float32)

    logits = hv[:, : h * w] + pb2_ref[...]
    mx = jnp.max(logits, axis=-1, keepdims=True)
    s = logits - mx
    lse = jnp.log(jnp.sum(jnp.exp(s), axis=-1, keepdims=True))
    prob_ref[...] = (s - lse).astype(prob_ref.dtype)

    v = jnp.maximum(hv[:, h * w: h * w + 64] + vb2_ref[...], 0.0)
    val = jnp.sum(v * vw3t_ref[...], axis=-1, keepdims=True) + vb3_ref[...]
    val_ref[...] = jnp.tanh(val).astype(val_ref.dtype)


def kernel(x_nchw, conv_w1, conv_w2, conv_w3, conv_b1, conv_b2, conv_b3,
           head_w, head_b, pw2, pb2, vw2, vb2, vw3, vb3):
    n, c, h, w = x_nchw.shape
    hw = h * w
    nb = next(cand for cand in (96, 32, 16, 8, 4, 2, 1) if n % cand == 0)
    bf = jnp.bfloat16

    x = x_nchw.reshape(n, c, hw)
    ident = jnp.eye(c, dtype=bf)

    cb1 = _band_weights(conv_w1, w)
    cb2 = _band_weights(conv_w2, w)
    cb3 = _band_weights(conv_w3, w)
    b1 = jnp.tile(conv_b1, (1, w))
    b2 = jnp.tile(conv_b2, (1, w))
    b3 = jnp.tile(conv_b3, (1, w))

    hd = jnp.zeros((w, 128, w, HEADC), jnp.float32)
    for i in range(w):
        hd = hd.at[i, :, i, :].set(head_w)
    hd = hd.reshape(w * 128, w * HEADC).astype(bf)
    hb = jnp.tile(head_b, (1, w))

    hw_out = pw2.shape[1]
    big = jnp.zeros((hw, HEADC, hw_out + 64), jnp.float32)
    big = big.at[:, :4, :hw_out].set(pw2.reshape(hw, 4, hw_out))
    big = big.at[:, 4:6, hw_out:].set(vw2.reshape(hw, 2, 64))
    wbig = big.reshape(hw * HEADC, hw_out + 64).astype(bf)

    vw3t = vw3.reshape(1, -1)

    fused = functools.partial(_fused_kernel, nb=nb, h=h, w=w)
    prob_out, val_out = pl.pallas_call(
        fused,
        out_shape=(jax.ShapeDtypeStruct((n, hw_out), jnp.float32),
                   jax.ShapeDtypeStruct((n, 1), jnp.float32)),
        grid=(n // nb,),
        in_specs=[
            pl.BlockSpec((nb, c, hw), lambda b: (b, 0, 0)),
            pl.BlockSpec(ident.shape, lambda b: (0, 0)),
            pl.BlockSpec(cb1.shape, lambda b: (0, 0, 0)),
            pl.BlockSpec(b1.shape, lambda b: (0, 0)),
            pl.BlockSpec(cb2.shape, lambda b: (0, 0, 0)),
            pl.BlockSpec(b2.shape, lambda b: (0, 0)),
            pl.BlockSpec(cb3.shape, lambda b: (0, 0, 0)),
            pl.BlockSpec(b3.shape, lambda b: (0, 0)),
            pl.BlockSpec(hd.shape, lambda b: (0, 0)),
            pl.BlockSpec(hb.shape, lambda b: (0, 0)),
            pl.BlockSpec(wbig.shape, lambda b: (0, 0)),
            pl.BlockSpec(pb2.shape, lambda b: (0, 0)),
            pl.BlockSpec(vb2.shape, lambda b: (0, 0)),
            pl.BlockSpec(vw3t.shape, lambda b: (0, 0)),
            pl.BlockSpec(vb3.shape, lambda b: (0, 0)),
        ],
        out_specs=(pl.BlockSpec((nb, hw_out), lambda b: (b, 0)),
                   pl.BlockSpec((nb, 1), lambda b: (b, 0))),
        scratch_shapes=[
            pltpu.VMEM((h + 2, nb, w * c), bf),
            pltpu.VMEM((h + 2, nb, w * 32), bf),
            pltpu.VMEM((h + 2, nb, w * 64), bf),
        ],
        compiler_params=pltpu.CompilerParams(
            dimension_semantics=("arbitrary",)),
    )(x, ident, cb1, b1, cb2, b2, cb3, b3, hd, hb, wbig, pb2, vb2, vw3t, vb3)
    return prob_out, val_out


# speedup vs baseline: 1.0931x; 1.0221x over previous
import functools

import jax
import jax.numpy as jnp
from jax.experimental import pallas as pl
from jax.experimental.pallas import tpu as pltpu

HEADC = 128


def _band_weights(wk, w):
    cin, cout = wk.shape[1], wk.shape[2]
    bands = jnp.zeros((3, w, cin, w, cout), wk.dtype)
    for dh in range(3):
        for dw in range(3):
            for wo in range(w):
                wi = wo + dw - 1
                if 0 <= wi < w:
                    bands = bands.at[dh, wi, :, wo, :].set(wk[dh * 3 + dw])
    return bands.reshape(3, w * cin, w * cout).astype(jnp.bfloat16)


def _fused_kernel(x_ref, cb1_ref, b1_ref, cb2_ref, b2_ref,
                  cb3_ref, b3_ref, hdw_ref, hdb_ref, wbig_ref, pb2_ref,
                  vb2_ref, vw3t_ref, vb3_ref, prob_ref, val_ref,
                  pad0, pad1, pad2, *, nb, h, w):
    mh = h * nb

    @pl.when(pl.program_id(0) == 0)
    def _():
        for ref in (pad0, pad1, pad2):
            ref[0, :, :] = jnp.zeros_like(ref[0, :, :])
            ref[h + 1, :, :] = jnp.zeros_like(ref[h + 1, :, :])

    def conv3x3_relu(src_ref, band_ref, b_ref):
        acc = None
        for dh in range(3):
            part = jnp.dot(src_ref[pl.ds(dh, h), :, :].reshape(mh, -1),
                           band_ref[dh], preferred_element_type=jnp.float32)
            acc = part if acc is None else acc + part
        return jnp.maximum(acc + b_ref[...], 0.0)

    xt = jnp.transpose(x_ref[...], (1, 0, 2, 3))
    pad0[pl.ds(1, h), :, :] = xt.reshape(h, nb, -1)
    y1 = conv3x3_relu(pad0, cb1_ref, b1_ref).astype(jnp.bfloat16)

    pad1[pl.ds(1, h), :, :] = y1.reshape(h, nb, -1)
    y2 = conv3x3_relu(pad1, cb2_ref, b2_ref).astype(jnp.bfloat16)

    pad2[pl.ds(1, h), :, :] = y2.reshape(h, nb, -1)
    y3 = conv3x3_relu(pad2, cb3_ref, b3_ref).astype(jnp.bfloat16)

    heads = jnp.dot(y3, hdw_ref[...], preferred_element_type=jnp.float32)
    heads = jnp.maximum(heads + hdb_ref[...], 0.0).astype(jnp.bfloat16)

    hs = jnp.swapaxes(heads.reshape(h, nb, w * HEADC), 0, 1)
    hv = jnp.dot(hs.reshape(nb, h * w * HEADC), wbig_ref[...],
                 preferred_element_type=jnp.float32)

    logits = hv[:, : h * w] + pb2_ref[...]
    mx = jnp.max(logits, axis=-1, keepdims=True)
    s = logits - mx
    lse = jnp.log(jnp.sum(jnp.exp(s), axis=-1, keepdims=True))
    prob_ref[...] = (s - lse).astype(prob_ref.dtype)

    v = jnp.maximum(hv[:, h * w: h * w + 64] + vb2_ref[...], 0.0)
    val = jnp.sum(v * vw3t_ref[...], axis=-1, keepdims=True) + vb3_ref[...]
    val_ref[...] = jnp.tanh(val).astype(val_ref.dtype)


def kernel(x_nchw, conv_w1, conv_w2, conv_w3, conv_b1, conv_b2, conv_b3,
           head_w, head_b, pw2, pb2, vw2, vb2, vw3, vb3):
    n, c, h, w = x_nchw.shape
    hw = h * w
    nb = next(cand for cand in (96, 32, 16, 8, 4, 2, 1) if n % cand == 0)
    bf = jnp.bfloat16

    x = jnp.einsum('ncp,cd->npd', x_nchw.reshape(n, c, hw),
                   jnp.eye(c, dtype=jnp.float32)).astype(bf)
    x = x.reshape(n, h, w, c)

    cb1 = _band_weights(conv_w1, w)
    cb2 = _band_weights(conv_w2, w)
    cb3 = _band_weights(conv_w3, w)
    b1 = jnp.tile(conv_b1, (1, w))
    b2 = jnp.tile(conv_b2, (1, w))
    b3 = jnp.tile(conv_b3, (1, w))

    hd = jnp.zeros((w, 128, w, HEADC), jnp.float32)
    for i in range(w):
        hd = hd.at[i, :, i, :].set(head_w)
    hd = hd.reshape(w * 128, w * HEADC).astype(bf)
    hb = jnp.tile(head_b, (1, w))

    hw_out = pw2.shape[1]
    big = jnp.zeros((hw, HEADC, hw_out + 64), jnp.float32)
    big = big.at[:, :4, :hw_out].set(pw2.reshape(hw, 4, hw_out))
    big = big.at[:, 4:6, hw_out:].set(vw2.reshape(hw, 2, 64))
    wbig = big.reshape(hw * HEADC, hw_out + 64).astype(bf)

    vw3t = vw3.reshape(1, -1)

    fused = functools.partial(_fused_kernel, nb=nb, h=h, w=w)
    prob_out, val_out = pl.pallas_call(
        fused,
        out_shape=(jax.ShapeDtypeStruct((n, hw_out), jnp.float32),
                   jax.ShapeDtypeStruct((n, 1), jnp.float32)),
        grid=(n // nb,),
        in_specs=[
            pl.BlockSpec((nb, h, w, c), lambda b: (b, 0, 0, 0)),
            pl.BlockSpec(cb1.shape, lambda b: (0, 0, 0)),
            pl.BlockSpec(b1.shape, lambda b: (0, 0)),
            pl.BlockSpec(cb2.shape, lambda b: (0, 0, 0)),
            pl.BlockSpec(b2.shape, lambda b: (0, 0)),
            pl.BlockSpec(cb3.shape, lambda b: (0, 0, 0)),
            pl.BlockSpec(b3.shape, lambda b: (0, 0)),
            pl.BlockSpec(hd.shape, lambda b: (0, 0)),
            pl.BlockSpec(hb.shape, lambda b: (0, 0)),
            pl.BlockSpec(wbig.shape, lambda b: (0, 0)),
            pl.BlockSpec(pb2.shape, lambda b: (0, 0)),
            pl.BlockSpec(vb2.shape, lambda b: (0, 0)),
            pl.BlockSpec(vw3t.shape, lambda b: (0, 0)),
            pl.BlockSpec(vb3.shape, lambda b: (0, 0)),
        ],
        out_specs=(pl.BlockSpec((nb, hw_out), lambda b: (b, 0)),
                   pl.BlockSpec((nb, 1), lambda b: (b, 0))),
        scratch_shapes=[
            pltpu.VMEM((h + 2, nb, w * c), bf),
            pltpu.VMEM((h + 2, nb, w * 32), bf),
            pltpu.VMEM((h + 2, nb, w * 64), bf),
        ],
        compiler_params=pltpu.CompilerParams(
            dimension_semantics=("arbitrary",)),
    )(x, cb1, b1, cb2, b2, cb3, b3, hd, hb, wbig, pb2, vb2, vw3t, vb3)
    return prob_out, val_out


# speedup vs baseline: 1.2540x; 1.1472x over previous
import functools

import jax
import jax.numpy as jnp
from jax.experimental import pallas as pl
from jax.experimental.pallas import tpu as pltpu

HEADC = 128


def _band_weights(wk, w):
    cin, cout = wk.shape[1], wk.shape[2]
    bands = jnp.zeros((3, w, cin, w, cout), wk.dtype)
    for dh in range(3):
        for dw in range(3):
            for wo in range(w):
                wi = wo + dw - 1
                if 0 <= wi < w:
                    bands = bands.at[dh, wi, :, wo, :].set(wk[dh * 3 + dw])
    return bands.reshape(3, w * cin, w * cout).astype(jnp.bfloat16)


def _fused_kernel(x_ref, cb1_ref, b1_ref, cb2_ref, b2_ref,
                  cb3_ref, b3_ref, hdw_ref, hdb_ref, wbig_ref, pb2_ref,
                  vb2_ref, vw3t_ref, vb3_ref, prob_ref, val_ref,
                  pad0, pad1, pad2, *, nb, h, w):
    mh = h * nb

    @pl.when(pl.program_id(0) == 0)
    def _():
        for ref in (pad0, pad1, pad2):
            ref[0, :, :] = jnp.zeros_like(ref[0, :, :])
            ref[h + 1, :, :] = jnp.zeros_like(ref[h + 1, :, :])

    def conv3x3_relu(src_ref, band_ref, b_ref):
        acc = None
        for dh in range(3):
            part = jnp.dot(src_ref[pl.ds(dh, h), :, :].reshape(mh, -1),
                           band_ref[dh], preferred_element_type=jnp.float32)
            acc = part if acc is None else acc + part
        return jnp.maximum(acc + b_ref[...], 0.0)

    xt = jnp.transpose(x_ref[...], (1, 0, 2, 3))
    pad0[pl.ds(1, h), :, :] = xt.reshape(h, nb, -1)
    y1 = conv3x3_relu(pad0, cb1_ref, b1_ref).astype(jnp.bfloat16)

    pad1[pl.ds(1, h), :, :] = y1.reshape(h, nb, -1)
    y2 = conv3x3_relu(pad1, cb2_ref, b2_ref).astype(jnp.bfloat16)

    pad2[pl.ds(1, h), :, :] = y2.reshape(h, nb, -1)
    y3 = conv3x3_relu(pad2, cb3_ref, b3_ref).astype(jnp.bfloat16)

    heads = jnp.dot(y3, hdw_ref[...], preferred_element_type=jnp.float32)
    heads = jnp.maximum(heads + hdb_ref[...], 0.0).astype(jnp.bfloat16)

    hs = jnp.swapaxes(heads.reshape(h, nb, w * HEADC), 0, 1)
    hv = jnp.dot(hs.reshape(nb, h * w * HEADC), wbig_ref[...],
                 preferred_element_type=jnp.float32)

    logits = hv[:, : h * w] + pb2_ref[...]
    mx = jnp.max(logits, axis=-1, keepdims=True)
    s = logits - mx
    lse = jnp.log(jnp.sum(jnp.exp(s), axis=-1, keepdims=True))
    prob_ref[...] = (s - lse).astype(prob_ref.dtype)

    v = jnp.maximum(hv[:, h * w: h * w + 64] + vb2_ref[...], 0.0)
    val = jnp.sum(v * vw3t_ref[...], axis=-1, keepdims=True) + vb3_ref[...]
    val_ref[...] = jnp.tanh(val).astype(val_ref.dtype)


def kernel(x_nchw, conv_w1, conv_w2, conv_w3, conv_b1, conv_b2, conv_b3,
           head_w, head_b, pw2, pb2, vw2, vb2, vw3, vb3):
    n, c, h, w = x_nchw.shape
    hw = h * w
    nb = next(cand for cand in (192, 96, 32, 16, 8, 4, 2, 1) if n % cand == 0)
    bf = jnp.bfloat16

    x = jnp.transpose(x_nchw, (0, 2, 3, 1)).astype(bf)

    cb1 = _band_weights(conv_w1, w)
    cb2 = _band_weights(conv_w2, w)
    cb3 = _band_weights(conv_w3, w)
    b1 = jnp.tile(conv_b1, (1, w))
    b2 = jnp.tile(conv_b2, (1, w))
    b3 = jnp.tile(conv_b3, (1, w))

    hd = jnp.zeros((w, 128, w, HEADC), jnp.float32)
    for i in range(w):
        hd = hd.at[i, :, i, :].set(head_w)
    hd = hd.reshape(w * 128, w * HEADC).astype(bf)
    hb = jnp.tile(head_b, (1, w))

    hw_out = pw2.shape[1]
    big = jnp.zeros((hw, HEADC, hw_out + 64), jnp.float32)
    big = big.at[:, :4, :hw_out].set(pw2.reshape(hw, 4, hw_out))
    big = big.at[:, 4:6, hw_out:].set(vw2.reshape(hw, 2, 64))
    wbig = big.reshape(hw * HEADC, hw_out + 64).astype(bf)

    vw3t = vw3.reshape(1, -1)

    fused = functools.partial(_fused_kernel, nb=nb, h=h, w=w)
    prob_out, val_out = pl.pallas_call(
        fused,
        out_shape=(jax.ShapeDtypeStruct((n, hw_out), jnp.float32),
                   jax.ShapeDtypeStruct((n, 1), jnp.float32)),
        grid=(n // nb,),
        in_specs=[
            pl.BlockSpec((nb, h, w, c), lambda b: (b, 0, 0, 0)),
            pl.BlockSpec(cb1.shape, lambda b: (0, 0, 0)),
            pl.BlockSpec(b1.shape, lambda b: (0, 0)),
            pl.BlockSpec(cb2.shape, lambda b: (0, 0, 0)),
            pl.BlockSpec(b2.shape, lambda b: (0, 0)),
            pl.BlockSpec(cb3.shape, lambda b: (0, 0, 0)),
            pl.BlockSpec(b3.shape, lambda b: (0, 0)),
            pl.BlockSpec(hd.shape, lambda b: (0, 0)),
            pl.BlockSpec(hb.shape, lambda b: (0, 0)),
            pl.BlockSpec(wbig.shape, lambda b: (0, 0)),
            pl.BlockSpec(pb2.shape, lambda b: (0, 0)),
            pl.BlockSpec(vb2.shape, lambda b: (0, 0)),
            pl.BlockSpec(vw3t.shape, lambda b: (0, 0)),
            pl.BlockSpec(vb3.shape, lambda b: (0, 0)),
        ],
        out_specs=(pl.BlockSpec((nb, hw_out), lambda b: (b, 0)),
                   pl.BlockSpec((nb, 1), lambda b: (b, 0))),
        scratch_shapes=[
            pltpu.VMEM((h + 2, nb, w * c), bf),
            pltpu.VMEM((h + 2, nb, w * 32), bf),
            pltpu.VMEM((h + 2, nb, w * 64), bf),
        ],
        compiler_params=pltpu.CompilerParams(
            dimension_semantics=("arbitrary",)),
    )(x, cb1, b1, cb2, b2, cb3, b3, hd, hb, wbig, pb2, vb2, vw3t, vb3)
    return prob_out, val_out
